# baseline (speedup 1.0000x reference)
import functools

import jax
import jax.numpy as jnp
from jax import lax
from jax.experimental import pallas as pl
from jax.experimental.pallas import tpu as pltpu
from jax.experimental.pallas import tpu_sc as plsc

_V = 10000
_E = 320000
_B = 8
_FIN = 128
_K = 3
_FOUT = 128

_NC = 2
_NS = 16
_LANES = 16

_NB = 80
_GB = 8
_NGR = 32
_BPT = _NGR * _GB
_EP = _NS * _BPT * _NB
_NBT = _EP // _NB
_RPT = 624
_RCH = 48
_NRCH = _RPT // _RCH
_REM = _V - _NS * _RPT
_REMBASE = _NS * _RPT
_PPC = _B // _NC


def _cheb_body(x0, cl, vl, rw, x1, x2,
               acc, obuf, xbuf,
               g0, g1, cla, clb, va, vb, ra, rb,
               sg0, sg1, sma, smb):
    c = lax.axis_index("c")
    s = lax.axis_index("s")
    gbase = s * _NGR

    _G = (g0, g1)
    _SG = (sg0, sg1)

    z16 = jnp.zeros((_LANES,), jnp.float32)

    def _zero_obuf():
        def _zrow(r, carry):
            for j in range(_FIN // _LANES):
                obuf[r, pl.ds(j * _LANES, _LANES)] = z16
            return carry

        lax.fori_loop(0, _RCH, _zrow, 0)

    def _issue_meta(gabs, cvr, vr, rr, sem):
        pltpu.async_copy(cl.at[pl.ds(gabs * _GB, _GB)], cvr, sem)
        pltpu.async_copy(vl.at[pl.ds(gabs * _GB, _GB)], vr, sem)
        pltpu.async_copy(rw.at[pl.ds(gabs * _GB, _GB)], rr, sem)

    def _wait_meta(cvr, vr, rr, sem):
        pltpu.make_async_copy(cl.at[pl.ds(0, _GB)], cvr, sem).wait()
        pltpu.make_async_copy(vl.at[pl.ds(0, _GB)], vr, sem).wait()
        pltpu.make_async_copy(rw.at[pl.ds(0, _GB)], rr, sem).wait()

    def _issue_gather(src_hbm, poff, cvr, kk, sl):
        pltpu.async_copy(
            src_hbm.at[pl.ds(poff, _V)].at[cvr.at[kk, pl.ds(0, _NB)]],
            _G[sl], _SG[sl])

    def _wait_gather(src_hbm, poff, cvr, kk, sl):
        pltpu.make_async_copy(
            src_hbm.at[pl.ds(poff, _V)].at[cvr.at[kk, pl.ds(0, _NB)]],
            _G[sl], _SG[sl]).wait()

    def _scale(vr, kk, sl):
        g = _G[sl]

        def _grp(grp, carry):
            v16 = vr[kk, pl.ds(grp * _LANES, _LANES)]
            for l in range(_LANES):
                e = grp * _LANES + l
                v = v16[l]
                for m in range(_FIN // _LANES):
                    sl2 = pl.ds(m * _LANES, _LANES)
                    g[e, sl2] = g[e, sl2] * v
            return carry

        lax.fori_loop(0, _NB // _LANES, _grp, 0)

    def _scatter(rr, kk, sl):
        pltpu.sync_copy(_G[sl], acc.at[rr.at[kk]], add=True)

    def _group(src_hbm, poff, cvr, vr, rr, ncvr, nvr, nrr, nsem, hn):
        def _kpair(kp, carry):
            k0 = 2 * kp
            k1 = 2 * kp + 1
            _wait_gather(src_hbm, poff, cvr, k0, 0)
            _issue_gather(src_hbm, poff, cvr, k1, 1)
            _scale(vr, k0, 0)
            _scatter(rr, k0, 0)

            _wait_gather(src_hbm, poff, cvr, k1, 1)

            @pl.when(kp < (_GB // 2) - 1)
            def _():
                _issue_gather(src_hbm, poff, cvr, k1 + 1, 0)

            @pl.when(jnp.logical_and(kp == (_GB // 2) - 1, hn))
            def _():
                _wait_meta(ncvr, nvr, nrr, nsem)
                _issue_gather(src_hbm, poff, ncvr, 0, 0)

            _scale(vr, k1, 1)
            _scatter(rr, k1, 1)
            return carry

        lax.fori_loop(0, _GB // 2, _kpair, 0)

    def _accumulate(src_hbm, poff):
        _issue_meta(gbase, cla, va, ra, sma)
        _wait_meta(cla, va, ra, sma)
        _issue_gather(src_hbm, poff, cla, 0, 0)

        def _upair(u, carry):
            _issue_meta(gbase + 2 * u + 1, clb, vb, rb, smb)
            _group(src_hbm, poff, cla, va, ra, clb, vb, rb, smb, True)

            @pl.when(u < (_NGR // 2) - 1)
            def _():
                _issue_meta(gbase + 2 * u + 2, cla, va, ra, sma)

            _group(src_hbm, poff, clb, vb, rb, cla, va, ra, sma,
                   u < (_NGR // 2) - 1)
            return carry

        lax.fori_loop(0, _NGR // 2, _upair, 0)

    def _zero_acc():
        _zero_obuf()
        for ci in range(_NRCH):
            pltpu.sync_copy(obuf, acc.at[pl.ds(s * _RPT + ci * _RCH, _RCH)])

        @pl.when(s == 0)
        def _():
            pltpu.sync_copy(obuf.at[pl.ds(0, _REM)],
                            acc.at[pl.ds(_REMBASE, _REM)])

    def _combine(nrows):
        def _comb(r, carry2):
            for j in range(_FIN // _LANES):
                sl = pl.ds(j * _LANES, _LANES)
                obuf[r, sl] = obuf[r, sl] * 2.0 - xbuf[r, sl]
            return carry2

        lax.fori_loop(0, nrows, _comb, 0)

    def _panel(q, carry):
        poff = (c * _PPC + q) * _V

        _zero_acc()
        plsc.subcore_barrier()
        _accumulate(x0, poff)
        plsc.subcore_barrier()
        for ci in range(_NRCH):
            r0 = s * _RPT + ci * _RCH
            pltpu.sync_copy(acc.at[pl.ds(r0, _RCH)],
                            x1.at[pl.ds(poff + r0, _RCH)])

        @pl.when(s == 0)
        def _():
            pltpu.sync_copy(acc.at[pl.ds(_REMBASE, _REM)],
                            x1.at[pl.ds(poff + _REMBASE, _REM)])

        plsc.subcore_barrier()

        _zero_acc()
        plsc.subcore_barrier()
        _accumulate(x1, poff)
        plsc.subcore_barrier()
        for ci in range(_NRCH):
            r0 = s * _RPT + ci * _RCH
            pltpu.sync_copy(acc.at[pl.ds(r0, _RCH)], obuf)
            pltpu.sync_copy(x0.at[pl.ds(poff + r0, _RCH)], xbuf)
            _combine(_RCH)
            pltpu.sync_copy(obuf, x2.at[pl.ds(poff + r0, _RCH)])

        @pl.when(s == 0)
        def _():
            pltpu.sync_copy(acc.at[pl.ds(_REMBASE, _REM)],
                            obuf.at[pl.ds(0, _REM)])
            pltpu.sync_copy(x0.at[pl.ds(poff + _REMBASE, _REM)],
                            xbuf.at[pl.ds(0, _REM)])
            _combine(_REM)
            pltpu.sync_copy(obuf.at[pl.ds(0, _REM)],
                            x2.at[pl.ds(poff + _REMBASE, _REM)])

        plsc.subcore_barrier()
        return carry

    lax.fori_loop(0, _PPC, _panel, 0)


@functools.cache
def _build_cheb_sc():
  return pl.kernel(
    _cheb_body,
    out_type=(jax.ShapeDtypeStruct((_B * _V, _FIN), jnp.float32),
              jax.ShapeDtypeStruct((_B * _V, _FIN), jnp.float32)),
    mesh=plsc.VectorSubcoreMesh(core_axis_name="c", subcore_axis_name="s",
                                num_cores=_NC, num_subcores=_NS),
    scratch_types=[
        pltpu.VMEM_SHARED((_V, _FIN), jnp.float32),
        pltpu.VMEM((_RCH, _FIN), jnp.float32),
        pltpu.VMEM((_RCH, _FIN), jnp.float32),
        pltpu.VMEM((_NB, _FIN), jnp.float32),
        pltpu.VMEM((_NB, _FIN), jnp.float32),
        pltpu.VMEM((_GB, _NB), jnp.int32),
        pltpu.VMEM((_GB, _NB), jnp.int32),
        pltpu.VMEM((_GB, _NB), jnp.float32),
        pltpu.VMEM((_GB, _NB), jnp.float32),
        pltpu.VMEM((_GB, _NB), jnp.int32),
        pltpu.VMEM((_GB, _NB), jnp.int32),
        pltpu.SemaphoreType.DMA,
        pltpu.SemaphoreType.DMA,
        pltpu.SemaphoreType.DMA,
        pltpu.SemaphoreType.DMA,
    ],
  )


_RB = 2000


def _mm_body(x0b, x1b, x2b, w0, w1, w2, bb, ob):
    acc = jnp.dot(x0b[...], w0[...], preferred_element_type=jnp.float32)
    acc = acc + jnp.dot(x1b[...], w1[...], preferred_element_type=jnp.float32)
    acc = acc + jnp.dot(x2b[...], w2[...], preferred_element_type=jnp.float32)
    ob[...] = acc + bb[...]


def _dense(x0, x1, x2, w0, w1, w2, bias2d):
    nblk = (_B * _V) // _RB
    row_spec = pl.BlockSpec((_RB, _FIN), lambda i: (i, 0))
    full_w = pl.BlockSpec((_FIN, _FOUT), lambda i: (0, 0))
    return pl.pallas_call(
        _mm_body,
        grid=(nblk,),
        in_specs=[row_spec, row_spec, row_spec, full_w, full_w, full_w,
                  pl.BlockSpec((1, _FOUT), lambda i: (0, 0))],
        out_specs=pl.BlockSpec((_RB, _FOUT), lambda i: (i, 0)),
        out_shape=jax.ShapeDtypeStruct((_B * _V, _FOUT), jnp.float32),
    )(x0, x1, x2, w0, w1, w2, bias2d)


def kernel(laplacian_indices, laplacian_values, inputs, weight, bias):
    rows = laplacian_indices[0]
    cols = laplacian_indices[1]
    pad = _EP - _E
    cl = jnp.concatenate([cols, jnp.zeros((pad,), cols.dtype)]).reshape(
        _NBT, _NB)
    rw = jnp.concatenate([rows, jnp.zeros((pad,), rows.dtype)]).reshape(
        _NBT, _NB)
    vl = jnp.concatenate(
        [laplacian_values, jnp.zeros((pad,), jnp.float32)]).reshape(
        _NBT, _NB)

    x0 = inputs.reshape(_B * _V, _FIN)
    x1, x2 = _build_cheb_sc()(x0, cl, vl, rw)
    w0 = weight[:, 0, :]
    w1 = weight[:, 1, :]
    w2 = weight[:, 2, :]
    out = _dense(x0, x1, x2, w0, w1, w2, bias.reshape(1, _FOUT))
    return out.reshape(_B, _V, _FOUT)


# speedup vs baseline: 1.0916x; 1.0916x over previous
import functools

import jax
import jax.numpy as jnp
from jax import lax
from jax.experimental import pallas as pl
from jax.experimental.pallas import tpu as pltpu
from jax.experimental.pallas import tpu_sc as plsc

_V = 10000
_E = 320000
_B = 8
_FIN = 128
_K = 3
_FOUT = 128

_NC = 2
_NS = 16
_LANES = 16

_NB = 128
_SB = 8
_NSUP = 20
_BPT = _NSUP * _SB
_EPT = _BPT * _NB
_EP = _NS * _EPT
_SBE = _SB * _NB
_RPT = 624
_RCH = 48
_NRCH = _RPT // _RCH
_REM = _V - _NS * _RPT
_REMBASE = _NS * _RPT
_PPC = _B // _NC


def _cheb_body(x0, cl, vl, rw, x1, x2,
               acc, obuf, xbuf,
               g0, g1, r0, r1, ca, cb, va, vb,
               sg0, sg1, sr0, sr1, sma, smb):
    c = lax.axis_index("c")
    s = lax.axis_index("s")
    ebase = s * _EPT

    _G = (g0, g1)
    _R = (r0, r1)
    _SG = (sg0, sg1)
    _SR = (sr0, sr1)

    z16 = jnp.zeros((_LANES,), jnp.float32)

    def _zero_obuf():
        def _zrow(r, carry):
            for j in range(_FIN // _LANES):
                obuf[r, pl.ds(j * _LANES, _LANES)] = z16
            return carry

        lax.fori_loop(0, _RCH, _zrow, 0)

    def _issue_meta(sup, cr, vr, sem):
        off = ebase + sup * _SBE
        pltpu.async_copy(cl.at[pl.ds(off, _SBE)], cr, sem)
        pltpu.async_copy(vl.at[pl.ds(off, _SBE)], vr, sem)

    def _wait_meta(cr, vr, sem):
        pltpu.make_async_copy(cl.at[pl.ds(0, _SBE)], cr, sem).wait()
        pltpu.make_async_copy(vl.at[pl.ds(0, _SBE)], vr, sem).wait()

    def _issue_rows(j, sl):
        pltpu.async_copy(rw.at[pl.ds(ebase + j * _NB, _NB)], _R[sl], _SR[sl])

    def _wait_rows(sl):
        pltpu.make_async_copy(rw.at[pl.ds(0, _NB)], _R[sl], _SR[sl]).wait()

    def _issue_gather(src_hbm, poff, cr, k, sl):
        pltpu.async_copy(
            src_hbm.at[pl.ds(poff, _V)].at[cr.at[pl.ds(k * _NB, _NB)]],
            _G[sl], _SG[sl])

    def _wait_gather(src_hbm, poff, cr, k, sl):
        pltpu.make_async_copy(
            src_hbm.at[pl.ds(poff, _V)].at[cr.at[pl.ds(k * _NB, _NB)]],
            _G[sl], _SG[sl]).wait()

    def _scale(vr, k, sl):
        g = _G[sl]

        def _grp(grp, carry):
            v16 = vr[pl.ds(k * _NB + grp * _LANES, _LANES)]
            for l in range(_LANES):
                e = grp * _LANES + l
                v = v16[l]
                for m in range(_FIN // _LANES):
                    sl2 = pl.ds(m * _LANES, _LANES)
                    g[e, sl2] = g[e, sl2] * v
            return carry

        lax.fori_loop(0, _NB // _LANES, _grp, 0)

    def _scatter(sl):
        pltpu.sync_copy(_G[sl], acc.at[_R[sl]], add=True)

    def _batch(src_hbm, poff, cr, vr, k, j, sl, ncr, nsem, last_sup,
               last_all):
        nsl = 1 - sl
        _wait_rows(sl)
        _wait_gather(src_hbm, poff, cr, k, sl)

        @pl.when(jnp.logical_not(last_all))
        def _():
            _issue_rows(j + 1, nsl)

        @pl.when(k + 1 < _SB)
        def _():
            _issue_gather(src_hbm, poff, cr, k + 1, nsl)

        @pl.when(jnp.logical_and(k + 1 == _SB, jnp.logical_not(last_sup)))
        def _():
            _wait_meta(ncr[0], ncr[1], nsem)
            _issue_gather(src_hbm, poff, ncr[0], 0, nsl)

        _scale(vr, k, sl)
        _scatter(sl)

    def _super(src_hbm, poff, cr, vr, ncr, nsem, sup, last_sup):
        def _kpair(kp, carry):
            j0 = sup * _SB + 2 * kp
            _batch(src_hbm, poff, cr, vr, 2 * kp, j0, 0, ncr, nsem,
                   last_sup, jnp.bool_(False))
            _batch(src_hbm, poff, cr, vr, 2 * kp + 1, j0 + 1, 1, ncr, nsem,
                   last_sup,
                   jnp.logical_and(last_sup, kp == _SB // 2 - 1))
            return carry

        lax.fori_loop(0, _SB // 2, _kpair, 0)

    def _accumulate(src_hbm, poff):
        _issue_meta(0, ca, va, sma)
        _wait_meta(ca, va, sma)
        _issue_rows(0, 0)
        _issue_gather(src_hbm, poff, ca, 0, 0)

        def _upair(u, carry):
            supa = 2 * u
            supb = 2 * u + 1
            _issue_meta(supb, cb, vb, smb)
            _super(src_hbm, poff, ca, va, (cb, vb), smb, supa,
                   jnp.bool_(False))

            @pl.when(u < _NSUP // 2 - 1)
            def _():
                _issue_meta(supb + 1, ca, va, sma)

            _super(src_hbm, poff, cb, vb, (ca, va), sma, supb,
                   u == _NSUP // 2 - 1)
            return carry

        lax.fori_loop(0, _NSUP // 2, _upair, 0)

    def _zero_acc():
        _zero_obuf()
        for ci in range(_NRCH):
            pltpu.sync_copy(obuf, acc.at[pl.ds(s * _RPT + ci * _RCH, _RCH)])

        @pl.when(s == 0)
        def _():
            pltpu.sync_copy(obuf.at[pl.ds(0, _REM)],
                            acc.at[pl.ds(_REMBASE, _REM)])

    def _combine(nrows):
        def _comb(r, carry2):
            for j in range(_FIN // _LANES):
                sl = pl.ds(j * _LANES, _LANES)
                obuf[r, sl] = obuf[r, sl] * 2.0 - xbuf[r, sl]
            return carry2

        lax.fori_loop(0, nrows, _comb, 0)

    def _panel(q, carry):
        poff = (c * _PPC + q) * _V

        _zero_acc()
        plsc.subcore_barrier()
        _accumulate(x0, poff)
        plsc.subcore_barrier()
        for ci in range(_NRCH):
            r0_ = s * _RPT + ci * _RCH
            pltpu.sync_copy(acc.at[pl.ds(r0_, _RCH)],
                            x1.at[pl.ds(poff + r0_, _RCH)])

        @pl.when(s == 0)
        def _():
            pltpu.sync_copy(acc.at[pl.ds(_REMBASE, _REM)],
                            x1.at[pl.ds(poff + _REMBASE, _REM)])

        plsc.subcore_barrier()

        _zero_acc()
        plsc.subcore_barrier()
        _accumulate(x1, poff)
        plsc.subcore_barrier()
        for ci in range(_NRCH):
            r0_ = s * _RPT + ci * _RCH
            pltpu.sync_copy(acc.at[pl.ds(r0_, _RCH)], obuf)
            pltpu.sync_copy(x0.at[pl.ds(poff + r0_, _RCH)], xbuf)
            _combine(_RCH)
            pltpu.sync_copy(obuf, x2.at[pl.ds(poff + r0_, _RCH)])

        @pl.when(s == 0)
        def _():
            pltpu.sync_copy(acc.at[pl.ds(_REMBASE, _REM)],
                            obuf.at[pl.ds(0, _REM)])
            pltpu.sync_copy(x0.at[pl.ds(poff + _REMBASE, _REM)],
                            xbuf.at[pl.ds(0, _REM)])
            _combine(_REM)
            pltpu.sync_copy(obuf.at[pl.ds(0, _REM)],
                            x2.at[pl.ds(poff + _REMBASE, _REM)])

        plsc.subcore_barrier()
        return carry

    lax.fori_loop(0, _PPC, _panel, 0)


@functools.cache
def _build_cheb_sc():
  return pl.kernel(
    _cheb_body,
    out_type=(jax.ShapeDtypeStruct((_B * _V, _FIN), jnp.float32),
              jax.ShapeDtypeStruct((_B * _V, _FIN), jnp.float32)),
    mesh=plsc.VectorSubcoreMesh(core_axis_name="c", subcore_axis_name="s",
                                num_cores=_NC, num_subcores=_NS),
    scratch_types=[
        pltpu.VMEM_SHARED((_V, _FIN), jnp.float32),
        pltpu.VMEM((_RCH, _FIN), jnp.float32),
        pltpu.VMEM((_RCH, _FIN), jnp.float32),
        pltpu.VMEM((_NB, _FIN), jnp.float32),
        pltpu.VMEM((_NB, _FIN), jnp.float32),
        pltpu.VMEM((_NB,), jnp.int32),
        pltpu.VMEM((_NB,), jnp.int32),
        pltpu.VMEM((_SBE,), jnp.int32),
        pltpu.VMEM((_SBE,), jnp.int32),
        pltpu.VMEM((_SBE,), jnp.float32),
        pltpu.VMEM((_SBE,), jnp.float32),
        pltpu.SemaphoreType.DMA,
        pltpu.SemaphoreType.DMA,
        pltpu.SemaphoreType.DMA,
        pltpu.SemaphoreType.DMA,
        pltpu.SemaphoreType.DMA,
        pltpu.SemaphoreType.DMA,
    ],
  )


_RB = 2000


def _mm_body(x0b, x1b, x2b, w0, w1, w2, bb, ob):
    acc = jnp.dot(x0b[...], w0[...], preferred_element_type=jnp.float32)
    acc = acc + jnp.dot(x1b[...], w1[...], preferred_element_type=jnp.float32)
    acc = acc + jnp.dot(x2b[...], w2[...], preferred_element_type=jnp.float32)
    ob[...] = acc + bb[...]


def _dense(x0, x1, x2, w0, w1, w2, bias2d):
    nblk = (_B * _V) // _RB
    row_spec = pl.BlockSpec((_RB, _FIN), lambda i: (i, 0))
    full_w = pl.BlockSpec((_FIN, _FOUT), lambda i: (0, 0))
    return pl.pallas_call(
        _mm_body,
        grid=(nblk,),
        in_specs=[row_spec, row_spec, row_spec, full_w, full_w, full_w,
                  pl.BlockSpec((1, _FOUT), lambda i: (0, 0))],
        out_specs=pl.BlockSpec((_RB, _FOUT), lambda i: (i, 0)),
        out_shape=jax.ShapeDtypeStruct((_B * _V, _FOUT), jnp.float32),
    )(x0, x1, x2, w0, w1, w2, bias2d)


def kernel(laplacian_indices, laplacian_values, inputs, weight, bias):
    rows = laplacian_indices[0]
    cols = laplacian_indices[1]
    pad = _EP - _E
    cl = jnp.concatenate([cols, jnp.zeros((pad,), cols.dtype)])
    rw = jnp.concatenate([rows, jnp.zeros((pad,), rows.dtype)])
    vl = jnp.concatenate([laplacian_values, jnp.zeros((pad,), jnp.float32)])

    x0 = inputs.reshape(_B * _V, _FIN)
    x1, x2 = _build_cheb_sc()(x0, cl, vl, rw)
    w0 = weight[:, 0, :]
    w1 = weight[:, 1, :]
    w2 = weight[:, 2, :]
    out = _dense(x0, x1, x2, w0, w1, w2, bias.reshape(1, _FOUT))
    return out.reshape(_B, _V, _FOUT)


# speedup vs baseline: 1.0918x; 1.0002x over previous
import functools

import jax
import jax.numpy as jnp
from jax import lax
from jax.experimental import pallas as pl
from jax.experimental.pallas import tpu as pltpu
from jax.experimental.pallas import tpu_sc as plsc

_V = 10000
_E = 320000
_B = 8
_FIN = 128
_K = 3
_FOUT = 128

_NC = 2
_NS = 16
_LANES = 16

_NB = 128
_SB = 8
_NSUP = 20
_BPT = _NSUP * _SB
_EPT = _BPT * _NB
_EP = _NS * _EPT
_SBE = _SB * _NB
_RPT = 624
_RCH = 48
_NRCH = _RPT // _RCH
_REM = _V - _NS * _RPT
_REMBASE = _NS * _RPT
_PPC = _B // _NC


def _cheb_body(x0, cl, vl, rw, x1, x2,
               acc, obuf, xbuf,
               g0, g1, c0, c1, r0, r1, va, vb,
               sg0, sg1, sc0, sc1, sr0, sr1, sma, smb):
    c = lax.axis_index("c")
    s = lax.axis_index("s")
    ebase = s * _EPT

    _G = (g0, g1)
    _C = (c0, c1)
    _R = (r0, r1)
    _SG = (sg0, sg1)
    _SC = (sc0, sc1)
    _SR = (sr0, sr1)

    z16 = jnp.zeros((_LANES,), jnp.float32)

    def _zero_obuf():
        def _zrow(r, carry):
            for j in range(_FIN // _LANES):
                obuf[r, pl.ds(j * _LANES, _LANES)] = z16
            return carry

        lax.fori_loop(0, _RCH, _zrow, 0)

    def _issue_vals(sup, vr, sem):
        off = ebase + sup * _SBE
        pltpu.async_copy(vl.at[pl.ds(off, _SBE)], vr, sem)

    def _wait_vals(vr, sem):
        pltpu.make_async_copy(vl.at[pl.ds(0, _SBE)], vr, sem).wait()

    def _issue_rows(j, sl):
        pltpu.async_copy(rw.at[pl.ds(ebase + j * _NB, _NB)], _R[sl], _SR[sl])

    def _wait_rows(sl):
        pltpu.make_async_copy(rw.at[pl.ds(0, _NB)], _R[sl], _SR[sl]).wait()

    def _issue_cols(j, sl):
        pltpu.async_copy(cl.at[pl.ds(ebase + j * _NB, _NB)], _C[sl], _SC[sl])

    def _wait_cols(sl):
        pltpu.make_async_copy(cl.at[pl.ds(0, _NB)], _C[sl], _SC[sl]).wait()

    def _issue_gather(src_hbm, poff, sl):
        pltpu.async_copy(src_hbm.at[pl.ds(poff, _V)].at[_C[sl]],
                         _G[sl], _SG[sl])

    def _wait_gather(src_hbm, poff, sl):
        pltpu.make_async_copy(src_hbm.at[pl.ds(poff, _V)].at[_C[sl]],
                              _G[sl], _SG[sl]).wait()

    def _scale(vr, k, sl):
        g = _G[sl]

        def _grp(grp, carry):
            v16 = vr[pl.ds(k * _NB + grp * _LANES, _LANES)]
            for l in range(_LANES):
                e = grp * _LANES + l
                v = v16[l]
                for m in range(_FIN // _LANES):
                    sl2 = pl.ds(m * _LANES, _LANES)
                    g[e, sl2] = g[e, sl2] * v
            return carry

        lax.fori_loop(0, _NB // _LANES, _grp, 0)

    def _scatter(sl):
        pltpu.sync_copy(_G[sl], acc.at[_R[sl]], add=True)

    def _batch(src_hbm, poff, vr, k, j, sl):
        nsl = 1 - sl
        _wait_rows(sl)
        _wait_gather(src_hbm, poff, sl)

        @pl.when(j + 1 < _BPT)
        def _():
            _wait_cols(nsl)
            _issue_gather(src_hbm, poff, nsl)
            _issue_rows(j + 1, nsl)

        @pl.when(j + 2 < _BPT)
        def _():
            _issue_cols(j + 2, sl)

        _scale(vr, k, sl)
        _scatter(sl)

    def _super(src_hbm, poff, vr, sem, sup):
        _wait_vals(vr, sem)

        def _kpair(kp, carry):
            j0 = sup * _SB + 2 * kp
            _batch(src_hbm, poff, vr, 2 * kp, j0, 0)
            _batch(src_hbm, poff, vr, 2 * kp + 1, j0 + 1, 1)
            return carry

        lax.fori_loop(0, _SB // 2, _kpair, 0)

    def _accumulate(src_hbm, poff):
        _issue_vals(0, va, sma)
        _issue_cols(0, 0)
        _wait_cols(0)
        _issue_gather(src_hbm, poff, 0)
        _issue_rows(0, 0)
        _issue_cols(1, 1)

        def _upair(u, carry):
            supa = 2 * u
            supb = 2 * u + 1
            _issue_vals(supb, vb, smb)
            _super(src_hbm, poff, va, sma, supa)

            @pl.when(u < _NSUP // 2 - 1)
            def _():
                _issue_vals(supb + 1, va, sma)

            _super(src_hbm, poff, vb, smb, supb)
            return carry

        lax.fori_loop(0, _NSUP // 2, _upair, 0)

    def _zero_acc():
        _zero_obuf()
        for ci in range(_NRCH):
            pltpu.sync_copy(obuf, acc.at[pl.ds(s * _RPT + ci * _RCH, _RCH)])

        @pl.when(s == 0)
        def _():
            pltpu.sync_copy(obuf.at[pl.ds(0, _REM)],
                            acc.at[pl.ds(_REMBASE, _REM)])

    def _combine(nrows):
        def _comb(r, carry2):
            for j in range(_FIN // _LANES):
                sl = pl.ds(j * _LANES, _LANES)
                obuf[r, sl] = obuf[r, sl] * 2.0 - xbuf[r, sl]
            return carry2

        lax.fori_loop(0, nrows, _comb, 0)

    def _panel(q, carry):
        poff = (c * _PPC + q) * _V

        _zero_acc()
        plsc.subcore_barrier()
        _accumulate(x0, poff)
        plsc.subcore_barrier()
        for ci in range(_NRCH):
            r0_ = s * _RPT + ci * _RCH
            pltpu.sync_copy(acc.at[pl.ds(r0_, _RCH)],
                            x1.at[pl.ds(poff + r0_, _RCH)])

        @pl.when(s == 0)
        def _():
            pltpu.sync_copy(acc.at[pl.ds(_REMBASE, _REM)],
                            x1.at[pl.ds(poff + _REMBASE, _REM)])

        plsc.subcore_barrier()

        _zero_acc()
        plsc.subcore_barrier()
        _accumulate(x1, poff)
        plsc.subcore_barrier()
        for ci in range(_NRCH):
            r0_ = s * _RPT + ci * _RCH
            pltpu.sync_copy(acc.at[pl.ds(r0_, _RCH)], obuf)
            pltpu.sync_copy(x0.at[pl.ds(poff + r0_, _RCH)], xbuf)
            _combine(_RCH)
            pltpu.sync_copy(obuf, x2.at[pl.ds(poff + r0_, _RCH)])

        @pl.when(s == 0)
        def _():
            pltpu.sync_copy(acc.at[pl.ds(_REMBASE, _REM)],
                            obuf.at[pl.ds(0, _REM)])
            pltpu.sync_copy(x0.at[pl.ds(poff + _REMBASE, _REM)],
                            xbuf.at[pl.ds(0, _REM)])
            _combine(_REM)
            pltpu.sync_copy(obuf.at[pl.ds(0, _REM)],
                            x2.at[pl.ds(poff + _REMBASE, _REM)])

        plsc.subcore_barrier()
        return carry

    lax.fori_loop(0, _PPC, _panel, 0)


@functools.cache
def _build_cheb_sc():
  return pl.kernel(
    _cheb_body,
    out_type=(jax.ShapeDtypeStruct((_B * _V, _FIN), jnp.float32),
              jax.ShapeDtypeStruct((_B * _V, _FIN), jnp.float32)),
    mesh=plsc.VectorSubcoreMesh(core_axis_name="c", subcore_axis_name="s",
                                num_cores=_NC, num_subcores=_NS),
    scratch_types=[
        pltpu.VMEM_SHARED((_V, _FIN), jnp.float32),
        pltpu.VMEM((_RCH, _FIN), jnp.float32),
        pltpu.VMEM((_RCH, _FIN), jnp.float32),
        pltpu.VMEM((_NB, _FIN), jnp.float32),
        pltpu.VMEM((_NB, _FIN), jnp.float32),
        pltpu.VMEM((_NB,), jnp.int32),
        pltpu.VMEM((_NB,), jnp.int32),
        pltpu.VMEM((_NB,), jnp.int32),
        pltpu.VMEM((_NB,), jnp.int32),
        pltpu.VMEM((_SBE,), jnp.float32),
        pltpu.VMEM((_SBE,), jnp.float32),
        pltpu.SemaphoreType.DMA,
        pltpu.SemaphoreType.DMA,
        pltpu.SemaphoreType.DMA,
        pltpu.SemaphoreType.DMA,
        pltpu.SemaphoreType.DMA,
        pltpu.SemaphoreType.DMA,
        pltpu.SemaphoreType.DMA,
        pltpu.SemaphoreType.DMA,
    ],
  )


_RB = 2000


def _mm_body(x0b, x1b, x2b, w0, w1, w2, bb, ob):
    acc = jnp.dot(x0b[...], w0[...], preferred_element_type=jnp.float32)
    acc = acc + jnp.dot(x1b[...], w1[...], preferred_element_type=jnp.float32)
    acc = acc + jnp.dot(x2b[...], w2[...], preferred_element_type=jnp.float32)
    ob[...] = acc + bb[...]


def _dense(x0, x1, x2, w0, w1, w2, bias2d):
    nblk = (_B * _V) // _RB
    row_spec = pl.BlockSpec((_RB, _FIN), lambda i: (i, 0))
    full_w = pl.BlockSpec((_FIN, _FOUT), lambda i: (0, 0))
    return pl.pallas_call(
        _mm_body,
        grid=(nblk,),
        in_specs=[row_spec, row_spec, row_spec, full_w, full_w, full_w,
                  pl.BlockSpec((1, _FOUT), lambda i: (0, 0))],
        out_specs=pl.BlockSpec((_RB, _FOUT), lambda i: (i, 0)),
        out_shape=jax.ShapeDtypeStruct((_B * _V, _FOUT), jnp.float32),
    )(x0, x1, x2, w0, w1, w2, bias2d)


def kernel(laplacian_indices, laplacian_values, inputs, weight, bias):
    rows = laplacian_indices[0]
    cols = laplacian_indices[1]
    pad = _EP - _E
    cl = jnp.concatenate([cols, jnp.zeros((pad,), cols.dtype)])
    rw = jnp.concatenate([rows, jnp.zeros((pad,), rows.dtype)])
    vl = jnp.concatenate([laplacian_values, jnp.zeros((pad,), jnp.float32)])

    x0 = inputs.reshape(_B * _V, _FIN)
    x1, x2 = _build_cheb_sc()(x0, cl, vl, rw)
    w0 = weight[:, 0, :]
    w1 = weight[:, 1, :]
    w2 = weight[:, 2, :]
    out = _dense(x0, x1, x2, w0, w1, w2, bias.reshape(1, _FOUT))
    return out.reshape(_B, _V, _FOUT)


# speedup vs baseline: 1.0971x; 1.0049x over previous
import functools

import jax
import jax.numpy as jnp
from jax import lax
from jax.experimental import pallas as pl
from jax.experimental.pallas import tpu as pltpu
from jax.experimental.pallas import tpu_sc as plsc

_V = 10000
_E = 320000
_B = 8
_FIN = 128
_K = 3
_FOUT = 128

_NC = 2
_NS = 16
_LANES = 16

_NB = 128
_SB = 8
_NSUP = 20
_BPT = _NSUP * _SB
_EPT = _BPT * _NB
_EP = _NS * _EPT
_SBE = _SB * _NB
_RPT = 624
_RCH = 48
_NRCH = _RPT // _RCH
_REM = _V - _NS * _RPT
_REMBASE = _NS * _RPT
_PPC = _B // _NC


def _cheb_body(x0, cl, vl, rw, x1, x2,
               acc, obuf, xbuf,
               g0, g1, c0, c1, r0, r1, v0, v1,
               sg0, sg1, sc0, sc1, srv0, srv1):
    c = lax.axis_index("c")
    s = lax.axis_index("s")
    ebase = s * _EPT

    _G = (g0, g1)
    _C = (c0, c1)
    _R = (r0, r1)
    _VV = (v0, v1)
    _SG = (sg0, sg1)
    _SC = (sc0, sc1)
    _SRV = (srv0, srv1)

    z16 = jnp.zeros((_LANES,), jnp.float32)

    def _zero_obuf():
        def _zrow(r, carry):
            for j in range(_FIN // _LANES):
                obuf[r, pl.ds(j * _LANES, _LANES)] = z16
            return carry

        lax.fori_loop(0, _RCH, _zrow, 0)

    def _issue_rv(j, sl):
        pltpu.async_copy(rw.at[pl.ds(ebase + j * _NB, _NB)],
                         _R[sl], _SRV[sl])
        pltpu.async_copy(vl.at[pl.ds(ebase + j * _NB, _NB)],
                         _VV[sl], _SRV[sl])

    def _wait_rv(sl):
        pltpu.make_async_copy(rw.at[pl.ds(0, _NB)], _R[sl], _SRV[sl]).wait()
        pltpu.make_async_copy(vl.at[pl.ds(0, _NB)], _VV[sl], _SRV[sl]).wait()

    def _issue_cols(j, sl):
        pltpu.async_copy(cl.at[pl.ds(ebase + j * _NB, _NB)], _C[sl], _SC[sl])

    def _wait_cols(sl):
        pltpu.make_async_copy(cl.at[pl.ds(0, _NB)], _C[sl], _SC[sl]).wait()

    def _issue_gather(src_hbm, poff, sl):
        pltpu.async_copy(src_hbm.at[pl.ds(poff, _V)].at[_C[sl]],
                         _G[sl], _SG[sl])

    def _wait_gather(src_hbm, poff, sl):
        pltpu.make_async_copy(src_hbm.at[pl.ds(poff, _V)].at[_C[sl]],
                              _G[sl], _SG[sl]).wait()

    def _scale(sl):
        g = _G[sl]
        vr = _VV[sl]

        def _grp(grp, carry):
            v16 = vr[pl.ds(grp * _LANES, _LANES)]
            for l in range(_LANES):
                e = grp * _LANES + l
                v = v16[l]
                for m in range(_FIN // _LANES):
                    sl2 = pl.ds(m * _LANES, _LANES)
                    g[e, sl2] = g[e, sl2] * v
            return carry

        lax.fori_loop(0, _NB // _LANES, _grp, 0)

    def _scatter(sl):
        pltpu.sync_copy(_G[sl], acc.at[_R[sl]], add=True)

    def _batch(src_hbm, poff, j, sl):
        nsl = 1 - sl
        _wait_rv(sl)
        _wait_gather(src_hbm, poff, sl)

        @pl.when(j + 1 < _BPT)
        def _():
            _wait_cols(nsl)
            _issue_gather(src_hbm, poff, nsl)
            _issue_rv(j + 1, nsl)

        @pl.when(j + 2 < _BPT)
        def _():
            _issue_cols(j + 2, sl)

        _scale(sl)
        _scatter(sl)

    def _accumulate(src_hbm, poff):
        _issue_cols(0, 0)
        _wait_cols(0)
        _issue_gather(src_hbm, poff, 0)
        _issue_rv(0, 0)
        _issue_cols(1, 1)

        def _pair(p, carry):
            _batch(src_hbm, poff, 2 * p, 0)
            _batch(src_hbm, poff, 2 * p + 1, 1)
            return carry

        lax.fori_loop(0, _BPT // 2, _pair, 0)

    def _zero_acc():
        _zero_obuf()
        for ci in range(_NRCH):
            pltpu.sync_copy(obuf, acc.at[pl.ds(s * _RPT + ci * _RCH, _RCH)])

        @pl.when(s == 0)
        def _():
            pltpu.sync_copy(obuf.at[pl.ds(0, _REM)],
                            acc.at[pl.ds(_REMBASE, _REM)])

    def _combine(nrows):
        def _comb(r, carry2):
            for j in range(_FIN // _LANES):
                sl = pl.ds(j * _LANES, _LANES)
                obuf[r, sl] = obuf[r, sl] * 2.0 - xbuf[r, sl]
            return carry2

        lax.fori_loop(0, nrows, _comb, 0)

    def _panel(q, carry):
        poff = (c * _PPC + q) * _V

        _zero_acc()
        plsc.subcore_barrier()
        _accumulate(x0, poff)
        plsc.subcore_barrier()
        for ci in range(_NRCH):
            r0_ = s * _RPT + ci * _RCH
            pltpu.sync_copy(acc.at[pl.ds(r0_, _RCH)],
                            x1.at[pl.ds(poff + r0_, _RCH)])

        @pl.when(s == 0)
        def _():
            pltpu.sync_copy(acc.at[pl.ds(_REMBASE, _REM)],
                            x1.at[pl.ds(poff + _REMBASE, _REM)])

        plsc.subcore_barrier()

        _zero_acc()
        plsc.subcore_barrier()
        _accumulate(x1, poff)
        plsc.subcore_barrier()
        for ci in range(_NRCH):
            r0_ = s * _RPT + ci * _RCH
            pltpu.sync_copy(acc.at[pl.ds(r0_, _RCH)], obuf)
            pltpu.sync_copy(x0.at[pl.ds(poff + r0_, _RCH)], xbuf)
            _combine(_RCH)
            pltpu.sync_copy(obuf, x2.at[pl.ds(poff + r0_, _RCH)])

        @pl.when(s == 0)
        def _():
            pltpu.sync_copy(acc.at[pl.ds(_REMBASE, _REM)],
                            obuf.at[pl.ds(0, _REM)])
            pltpu.sync_copy(x0.at[pl.ds(poff + _REMBASE, _REM)],
                            xbuf.at[pl.ds(0, _REM)])
            _combine(_REM)
            pltpu.sync_copy(obuf.at[pl.ds(0, _REM)],
                            x2.at[pl.ds(poff + _REMBASE, _REM)])

        plsc.subcore_barrier()
        return carry

    lax.fori_loop(0, _PPC, _panel, 0)


@functools.cache
def _build_cheb_sc():
  return pl.kernel(
    _cheb_body,
    out_type=(jax.ShapeDtypeStruct((_B * _V, _FIN), jnp.float32),
              jax.ShapeDtypeStruct((_B * _V, _FIN), jnp.float32)),
    mesh=plsc.VectorSubcoreMesh(core_axis_name="c", subcore_axis_name="s",
                                num_cores=_NC, num_subcores=_NS),
    scratch_types=[
        pltpu.VMEM_SHARED((_V, _FIN), jnp.float32),
        pltpu.VMEM((_RCH, _FIN), jnp.float32),
        pltpu.VMEM((_RCH, _FIN), jnp.float32),
        pltpu.VMEM((_NB, _FIN), jnp.float32),
        pltpu.VMEM((_NB, _FIN), jnp.float32),
        pltpu.VMEM((_NB,), jnp.int32),
        pltpu.VMEM((_NB,), jnp.int32),
        pltpu.VMEM((_NB,), jnp.int32),
        pltpu.VMEM((_NB,), jnp.int32),
        pltpu.VMEM((_NB,), jnp.float32),
        pltpu.VMEM((_NB,), jnp.float32),
        pltpu.SemaphoreType.DMA,
        pltpu.SemaphoreType.DMA,
        pltpu.SemaphoreType.DMA,
        pltpu.SemaphoreType.DMA,
        pltpu.SemaphoreType.DMA,
        pltpu.SemaphoreType.DMA,
    ],
  )


_RB = 2000


def _mm_body(x0b, x1b, x2b, w0, w1, w2, bb, ob):
    acc = jnp.dot(x0b[...], w0[...], preferred_element_type=jnp.float32)
    acc = acc + jnp.dot(x1b[...], w1[...], preferred_element_type=jnp.float32)
    acc = acc + jnp.dot(x2b[...], w2[...], preferred_element_type=jnp.float32)
    ob[...] = acc + bb[...]


def _dense(x0, x1, x2, w0, w1, w2, bias2d):
    nblk = (_B * _V) // _RB
    row_spec = pl.BlockSpec((_RB, _FIN), lambda i: (i, 0))
    full_w = pl.BlockSpec((_FIN, _FOUT), lambda i: (0, 0))
    return pl.pallas_call(
        _mm_body,
        grid=(nblk,),
        in_specs=[row_spec, row_spec, row_spec, full_w, full_w, full_w,
                  pl.BlockSpec((1, _FOUT), lambda i: (0, 0))],
        out_specs=pl.BlockSpec((_RB, _FOUT), lambda i: (i, 0)),
        out_shape=jax.ShapeDtypeStruct((_B * _V, _FOUT), jnp.float32),
    )(x0, x1, x2, w0, w1, w2, bias2d)


def kernel(laplacian_indices, laplacian_values, inputs, weight, bias):
    rows = laplacian_indices[0]
    cols = laplacian_indices[1]
    pad = _EP - _E
    cl = jnp.concatenate([cols, jnp.zeros((pad,), cols.dtype)])
    rw = jnp.concatenate([rows, jnp.zeros((pad,), rows.dtype)])
    vl = jnp.concatenate([laplacian_values, jnp.zeros((pad,), jnp.float32)])

    x0 = inputs.reshape(_B * _V, _FIN)
    x1, x2 = _build_cheb_sc()(x0, cl, vl, rw)
    w0 = weight[:, 0, :]
    w1 = weight[:, 1, :]
    w2 = weight[:, 2, :]
    out = _dense(x0, x1, x2, w0, w1, w2, bias.reshape(1, _FOUT))
    return out.reshape(_B, _V, _FOUT)


# speedup vs baseline: 1.7825x; 1.6248x over previous
import functools

import jax
import jax.numpy as jnp
from jax import lax
from jax.experimental import pallas as pl
from jax.experimental.pallas import tpu as pltpu
from jax.experimental.pallas import tpu_sc as plsc

_V = 10000
_E = 320000
_B = 8
_FIN = 128
_K = 3
_FOUT = 128

_NC = 2
_NS = 16
_LANES = 16

_NB = 120
_BPT = 168
_EPT = _BPT * _NB
_EP = _NS * _EPT
_RPT = 624
_RCH = 48
_NRCH = _RPT // _RCH
_REM = _V - _NS * _RPT
_REMBASE = _NS * _RPT
_PPC = _B // _NC


def _cheb_body(x0, cl, vl, rw, x1, x2,
               acc, obuf, xbuf,
               g0, g1, c0, c1, r0, r1, v0, v1,
               sg0, sg1, sc0, sc1, srv0, srv1):
    c = lax.axis_index("c")
    s = lax.axis_index("s")
    ebase = s * _EPT

    _G = (g0, g1)
    _C = (c0, c1)
    _R = (r0, r1)
    _VV = (v0, v1)
    _SG = (sg0, sg1)
    _SC = (sc0, sc1)
    _SRV = (srv0, srv1)

    z16 = jnp.zeros((_LANES,), jnp.float32)

    def _zero_obuf():
        def _zrow(r, carry):
            for j in range(_FIN // _LANES):
                obuf[r, pl.ds(j * _LANES, _LANES)] = z16
            return carry

        lax.fori_loop(0, _RCH, _zrow, 0)

    def _issue_rv(j, sl):
        pltpu.async_copy(rw.at[pl.ds(ebase + j * _NB, _NB)],
                         _R[sl], _SRV[sl])
        pltpu.async_copy(vl.at[pl.ds(ebase + j * _NB, _NB)],
                         _VV[sl], _SRV[sl])

    def _wait_rv(sl):
        pltpu.make_async_copy(rw.at[pl.ds(0, _NB)], _R[sl], _SRV[sl]).wait()
        pltpu.make_async_copy(vl.at[pl.ds(0, _NB)], _VV[sl], _SRV[sl]).wait()

    def _issue_cols(j, sl):
        pltpu.async_copy(cl.at[pl.ds(ebase + j * _NB, _NB)], _C[sl], _SC[sl])

    def _wait_cols(sl):
        pltpu.make_async_copy(cl.at[pl.ds(0, _NB)], _C[sl], _SC[sl]).wait()

    def _issue_gather(src_hbm, poff, sl):
        pltpu.async_copy(src_hbm.at[pl.ds(poff, _V)].at[_C[sl]],
                         _G[sl], _SG[sl])

    def _wait_gather(src_hbm, poff, sl):
        pltpu.make_async_copy(src_hbm.at[pl.ds(poff, _V)].at[_C[sl]],
                              _G[sl], _SG[sl]).wait()

    def _scale(sl):
        g = _G[sl]
        vr = _VV[sl]

        def _grp(grp, carry):
            v16 = vr[pl.ds(grp * _LANES, _LANES)]
            for l in range(_LANES):
                e = grp * _LANES + l
                v = v16[l]
                for m in range(_FIN // _LANES):
                    sl2 = pl.ds(m * _LANES, _LANES)
                    g[e, sl2] = g[e, sl2] * v
            return carry

        lax.fori_loop(0, _NB // _LANES, _grp, 0)

    def _scatter(sl):
        pltpu.sync_copy(_G[sl], acc.at[_R[sl]], add=True)

    def _batch(src_hbm, poff, j, sl):
        nsl = 1 - sl
        _wait_rv(sl)
        _wait_gather(src_hbm, poff, sl)

        @pl.when(j + 1 < _BPT)
        def _():
            _wait_cols(nsl)
            _issue_gather(src_hbm, poff, nsl)
            _issue_rv(j + 1, nsl)

        @pl.when(j + 2 < _BPT)
        def _():
            _issue_cols(j + 2, sl)

        _scale(sl)
        _scatter(sl)

    def _accumulate(src_hbm, poff):
        _issue_cols(0, 0)
        _wait_cols(0)
        _issue_gather(src_hbm, poff, 0)
        _issue_rv(0, 0)
        _issue_cols(1, 1)

        def _pair(p, carry):
            _batch(src_hbm, poff, 2 * p, 0)
            _batch(src_hbm, poff, 2 * p + 1, 1)
            return carry

        lax.fori_loop(0, _BPT // 2, _pair, 0)

    def _zero_acc():
        _zero_obuf()
        for ci in range(_NRCH):
            pltpu.sync_copy(obuf, acc.at[pl.ds(s * _RPT + ci * _RCH, _RCH)])

        @pl.when(s == 0)
        def _():
            pltpu.sync_copy(obuf.at[pl.ds(0, _REM)],
                            acc.at[pl.ds(_REMBASE, _REM)])

    def _combine(nrows):
        def _comb(r, carry2):
            for j in range(_FIN // _LANES):
                sl = pl.ds(j * _LANES, _LANES)
                obuf[r, sl] = obuf[r, sl] * 2.0 - xbuf[r, sl]
            return carry2

        lax.fori_loop(0, nrows, _comb, 0)

    def _panel(q, carry):
        poff = (c * _PPC + q) * _V

        _zero_acc()
        plsc.subcore_barrier()
        _accumulate(x0, poff)
        plsc.subcore_barrier()
        for ci in range(_NRCH):
            r0_ = s * _RPT + ci * _RCH
            pltpu.sync_copy(acc.at[pl.ds(r0_, _RCH)],
                            x1.at[pl.ds(poff + r0_, _RCH)])

        @pl.when(s == 0)
        def _():
            pltpu.sync_copy(acc.at[pl.ds(_REMBASE, _REM)],
                            x1.at[pl.ds(poff + _REMBASE, _REM)])

        plsc.subcore_barrier()

        _zero_acc()
        plsc.subcore_barrier()
        _accumulate(x1, poff)
        plsc.subcore_barrier()
        for ci in range(_NRCH):
            r0_ = s * _RPT + ci * _RCH
            pltpu.sync_copy(acc.at[pl.ds(r0_, _RCH)], obuf)
            pltpu.sync_copy(x0.at[pl.ds(poff + r0_, _RCH)], xbuf)
            _combine(_RCH)
            pltpu.sync_copy(obuf, x2.at[pl.ds(poff + r0_, _RCH)])

        @pl.when(s == 0)
        def _():
            pltpu.sync_copy(acc.at[pl.ds(_REMBASE, _REM)],
                            obuf.at[pl.ds(0, _REM)])
            pltpu.sync_copy(x0.at[pl.ds(poff + _REMBASE, _REM)],
                            xbuf.at[pl.ds(0, _REM)])
            _combine(_REM)
            pltpu.sync_copy(obuf.at[pl.ds(0, _REM)],
                            x2.at[pl.ds(poff + _REMBASE, _REM)])

        plsc.subcore_barrier()
        return carry

    lax.fori_loop(0, _PPC, _panel, 0)


@functools.cache
def _build_cheb_sc():
  return pl.kernel(
    _cheb_body,
    out_type=(jax.ShapeDtypeStruct((_B * _V, _FIN), jnp.float32),
              jax.ShapeDtypeStruct((_B * _V, _FIN), jnp.float32)),
    mesh=plsc.VectorSubcoreMesh(core_axis_name="c", subcore_axis_name="s",
                                num_cores=_NC, num_subcores=_NS),
    scratch_types=[
        pltpu.VMEM_SHARED((_V, _FIN), jnp.float32),
        pltpu.VMEM((_RCH, _FIN), jnp.float32),
        pltpu.VMEM((_RCH, _FIN), jnp.float32),
        pltpu.VMEM((_NB, _FIN), jnp.float32),
        pltpu.VMEM((_NB, _FIN), jnp.float32),
        pltpu.VMEM((_NB,), jnp.int32),
        pltpu.VMEM((_NB,), jnp.int32),
        pltpu.VMEM((_NB,), jnp.int32),
        pltpu.VMEM((_NB,), jnp.int32),
        pltpu.VMEM((_NB,), jnp.float32),
        pltpu.VMEM((_NB,), jnp.float32),
        pltpu.SemaphoreType.DMA,
        pltpu.SemaphoreType.DMA,
        pltpu.SemaphoreType.DMA,
        pltpu.SemaphoreType.DMA,
        pltpu.SemaphoreType.DMA,
        pltpu.SemaphoreType.DMA,
    ],
  )


_RB = 2000


def _mm_body(x0b, x1b, x2b, w0, w1, w2, bb, ob):
    acc = jnp.dot(x0b[...], w0[...], preferred_element_type=jnp.float32)
    acc = acc + jnp.dot(x1b[...], w1[...], preferred_element_type=jnp.float32)
    acc = acc + jnp.dot(x2b[...], w2[...], preferred_element_type=jnp.float32)
    ob[...] = acc + bb[...]


def _dense(x0, x1, x2, w0, w1, w2, bias2d):
    nblk = (_B * _V) // _RB
    row_spec = pl.BlockSpec((_RB, _FIN), lambda i: (i, 0))
    full_w = pl.BlockSpec((_FIN, _FOUT), lambda i: (0, 0))
    return pl.pallas_call(
        _mm_body,
        grid=(nblk,),
        in_specs=[row_spec, row_spec, row_spec, full_w, full_w, full_w,
                  pl.BlockSpec((1, _FOUT), lambda i: (0, 0))],
        out_specs=pl.BlockSpec((_RB, _FOUT), lambda i: (i, 0)),
        out_shape=jax.ShapeDtypeStruct((_B * _V, _FOUT), jnp.float32),
    )(x0, x1, x2, w0, w1, w2, bias2d)


def kernel(laplacian_indices, laplacian_values, inputs, weight, bias):
    rows = laplacian_indices[0]
    cols = laplacian_indices[1]
    pad = _EP - _E
    cl = jnp.concatenate([cols, jnp.zeros((pad,), cols.dtype)])
    rw = jnp.concatenate([rows, jnp.zeros((pad,), rows.dtype)])
    vl = jnp.concatenate([laplacian_values, jnp.zeros((pad,), jnp.float32)])

    x0 = inputs.reshape(_B * _V, _FIN)
    x1, x2 = _build_cheb_sc()(x0, cl, vl, rw)
    w0 = weight[:, 0, :]
    w1 = weight[:, 1, :]
    w2 = weight[:, 2, :]
    out = _dense(x0, x1, x2, w0, w1, w2, bias.reshape(1, _FOUT))
    return out.reshape(_B, _V, _FOUT)


# speedup vs baseline: 2.2009x; 1.2347x over previous
import functools

import jax
import jax.numpy as jnp
from jax import lax
from jax.experimental import pallas as pl
from jax.experimental.pallas import tpu as pltpu
from jax.experimental.pallas import tpu_sc as plsc

_V = 10000
_E = 320000
_B = 8
_FIN = 128
_K = 3
_FOUT = 128

_NC = 2
_NS = 16
_LANES = 16

_NB = 80
_BPT = 250
_EPT = _BPT * _NB
_EP = _NS * _EPT
_RPT = 624
_RCH = 48
_NRCH = _RPT // _RCH
_REM = _V - _NS * _RPT
_REMBASE = _NS * _RPT
_PPC = _B // _NC


def _cheb_body(x0, cl, vl, rw, x1, x2,
               acc, obuf, xbuf,
               g0, g1, c0, c1, r0, r1, v0, v1,
               sg0, sg1, sc0, sc1, srv0, srv1):
    c = lax.axis_index("c")
    s = lax.axis_index("s")
    ebase = s * _EPT

    _G = (g0, g1)
    _C = (c0, c1)
    _R = (r0, r1)
    _VV = (v0, v1)
    _SG = (sg0, sg1)
    _SC = (sc0, sc1)
    _SRV = (srv0, srv1)

    z16 = jnp.zeros((_LANES,), jnp.float32)

    def _zero_obuf():
        def _zrow(r, carry):
            for j in range(_FIN // _LANES):
                obuf[r, pl.ds(j * _LANES, _LANES)] = z16
            return carry

        lax.fori_loop(0, _RCH, _zrow, 0)

    def _issue_rv(j, sl):
        pltpu.async_copy(rw.at[pl.ds(ebase + j * _NB, _NB)],
                         _R[sl], _SRV[sl])
        pltpu.async_copy(vl.at[pl.ds(ebase + j * _NB, _NB)],
                         _VV[sl], _SRV[sl])

    def _wait_rv(sl):
        pltpu.make_async_copy(rw.at[pl.ds(0, _NB)], _R[sl], _SRV[sl]).wait()
        pltpu.make_async_copy(vl.at[pl.ds(0, _NB)], _VV[sl], _SRV[sl]).wait()

    def _issue_cols(j, sl):
        pltpu.async_copy(cl.at[pl.ds(ebase + j * _NB, _NB)], _C[sl], _SC[sl])

    def _wait_cols(sl):
        pltpu.make_async_copy(cl.at[pl.ds(0, _NB)], _C[sl], _SC[sl]).wait()

    def _issue_gather(src_hbm, poff, sl):
        pltpu.async_copy(src_hbm.at[pl.ds(poff, _V)].at[_C[sl]],
                         _G[sl], _SG[sl])

    def _wait_gather(src_hbm, poff, sl):
        pltpu.make_async_copy(src_hbm.at[pl.ds(poff, _V)].at[_C[sl]],
                              _G[sl], _SG[sl]).wait()

    def _scale(sl):
        g = _G[sl]
        vr = _VV[sl]

        def _grp(grp, carry):
            v16 = vr[pl.ds(grp * _LANES, _LANES)]
            for l in range(_LANES):
                e = grp * _LANES + l
                v = v16[l]
                for m in range(_FIN // _LANES):
                    sl2 = pl.ds(m * _LANES, _LANES)
                    g[e, sl2] = g[e, sl2] * v
            return carry

        lax.fori_loop(0, _NB // _LANES, _grp, 0)

    def _scatter(sl):
        pltpu.sync_copy(_G[sl], acc.at[_R[sl]], add=True)

    def _batch(src_hbm, poff, j, sl):
        nsl = 1 - sl
        _wait_rv(sl)
        _wait_gather(src_hbm, poff, sl)

        @pl.when(j + 1 < _BPT)
        def _():
            _wait_cols(nsl)
            _issue_gather(src_hbm, poff, nsl)
            _issue_rv(j + 1, nsl)

        @pl.when(j + 2 < _BPT)
        def _():
            _issue_cols(j + 2, sl)

        _scale(sl)
        _scatter(sl)

    def _accumulate(src_hbm, poff):
        _issue_cols(0, 0)
        _wait_cols(0)
        _issue_gather(src_hbm, poff, 0)
        _issue_rv(0, 0)
        _issue_cols(1, 1)

        def _pair(p, carry):
            _batch(src_hbm, poff, 2 * p, 0)
            _batch(src_hbm, poff, 2 * p + 1, 1)
            return carry

        lax.fori_loop(0, _BPT // 2, _pair, 0)

    def _zero_acc():
        _zero_obuf()
        for ci in range(_NRCH):
            pltpu.sync_copy(obuf, acc.at[pl.ds(s * _RPT + ci * _RCH, _RCH)])

        @pl.when(s == 0)
        def _():
            pltpu.sync_copy(obuf.at[pl.ds(0, _REM)],
                            acc.at[pl.ds(_REMBASE, _REM)])

    def _combine(nrows):
        def _comb(r, carry2):
            for j in range(_FIN // _LANES):
                sl = pl.ds(j * _LANES, _LANES)
                obuf[r, sl] = obuf[r, sl] * 2.0 - xbuf[r, sl]
            return carry2

        lax.fori_loop(0, nrows, _comb, 0)

    def _panel(q, carry):
        poff = (c * _PPC + q) * _V

        _zero_acc()
        plsc.subcore_barrier()
        _accumulate(x0, poff)
        plsc.subcore_barrier()
        for ci in range(_NRCH):
            r0_ = s * _RPT + ci * _RCH
            pltpu.sync_copy(acc.at[pl.ds(r0_, _RCH)],
                            x1.at[pl.ds(poff + r0_, _RCH)])

        @pl.when(s == 0)
        def _():
            pltpu.sync_copy(acc.at[pl.ds(_REMBASE, _REM)],
                            x1.at[pl.ds(poff + _REMBASE, _REM)])

        plsc.subcore_barrier()

        _zero_acc()
        plsc.subcore_barrier()
        _accumulate(x1, poff)
        plsc.subcore_barrier()
        for ci in range(_NRCH):
            r0_ = s * _RPT + ci * _RCH
            pltpu.sync_copy(acc.at[pl.ds(r0_, _RCH)], obuf)
            pltpu.sync_copy(x0.at[pl.ds(poff + r0_, _RCH)], xbuf)
            _combine(_RCH)
            pltpu.sync_copy(obuf, x2.at[pl.ds(poff + r0_, _RCH)])

        @pl.when(s == 0)
        def _():
            pltpu.sync_copy(acc.at[pl.ds(_REMBASE, _REM)],
                            obuf.at[pl.ds(0, _REM)])
            pltpu.sync_copy(x0.at[pl.ds(poff + _REMBASE, _REM)],
                            xbuf.at[pl.ds(0, _REM)])
            _combine(_REM)
            pltpu.sync_copy(obuf.at[pl.ds(0, _REM)],
                            x2.at[pl.ds(poff + _REMBASE, _REM)])

        plsc.subcore_barrier()
        return carry

    lax.fori_loop(0, _PPC, _panel, 0)


@functools.cache
def _build_cheb_sc():
  return pl.kernel(
    _cheb_body,
    out_type=(jax.ShapeDtypeStruct((_B * _V, _FIN), jnp.float32),
              jax.ShapeDtypeStruct((_B * _V, _FIN), jnp.float32)),
    mesh=plsc.VectorSubcoreMesh(core_axis_name="c", subcore_axis_name="s",
                                num_cores=_NC, num_subcores=_NS),
    scratch_types=[
        pltpu.VMEM_SHARED((_V, _FIN), jnp.float32),
        pltpu.VMEM((_RCH, _FIN), jnp.float32),
        pltpu.VMEM((_RCH, _FIN), jnp.float32),
        pltpu.VMEM((_NB, _FIN), jnp.float32),
        pltpu.VMEM((_NB, _FIN), jnp.float32),
        pltpu.VMEM((_NB,), jnp.int32),
        pltpu.VMEM((_NB,), jnp.int32),
        pltpu.VMEM((_NB,), jnp.int32),
        pltpu.VMEM((_NB,), jnp.int32),
        pltpu.VMEM((_NB,), jnp.float32),
        pltpu.VMEM((_NB,), jnp.float32),
        pltpu.SemaphoreType.DMA,
        pltpu.SemaphoreType.DMA,
        pltpu.SemaphoreType.DMA,
        pltpu.SemaphoreType.DMA,
        pltpu.SemaphoreType.DMA,
        pltpu.SemaphoreType.DMA,
    ],
  )


_RB = 2000


def _mm_body(x0b, x1b, x2b, w0, w1, w2, bb, ob):
    acc = jnp.dot(x0b[...], w0[...], preferred_element_type=jnp.float32)
    acc = acc + jnp.dot(x1b[...], w1[...], preferred_element_type=jnp.float32)
    acc = acc + jnp.dot(x2b[...], w2[...], preferred_element_type=jnp.float32)
    ob[...] = acc + bb[...]


def _dense(x0, x1, x2, w0, w1, w2, bias2d):
    nblk = (_B * _V) // _RB
    row_spec = pl.BlockSpec((_RB, _FIN), lambda i: (i, 0))
    full_w = pl.BlockSpec((_FIN, _FOUT), lambda i: (0, 0))
    return pl.pallas_call(
        _mm_body,
        grid=(nblk,),
        in_specs=[row_spec, row_spec, row_spec, full_w, full_w, full_w,
                  pl.BlockSpec((1, _FOUT), lambda i: (0, 0))],
        out_specs=pl.BlockSpec((_RB, _FOUT), lambda i: (i, 0)),
        out_shape=jax.ShapeDtypeStruct((_B * _V, _FOUT), jnp.float32),
    )(x0, x1, x2, w0, w1, w2, bias2d)


def kernel(laplacian_indices, laplacian_values, inputs, weight, bias):
    rows = laplacian_indices[0]
    cols = laplacian_indices[1]
    pad = _EP - _E
    cl = jnp.concatenate([cols, jnp.zeros((pad,), cols.dtype)])
    rw = jnp.concatenate([rows, jnp.zeros((pad,), rows.dtype)])
    vl = jnp.concatenate([laplacian_values, jnp.zeros((pad,), jnp.float32)])

    x0 = inputs.reshape(_B * _V, _FIN)
    x1, x2 = _build_cheb_sc()(x0, cl, vl, rw)
    w0 = weight[:, 0, :]
    w1 = weight[:, 1, :]
    w2 = weight[:, 2, :]
    out = _dense(x0, x1, x2, w0, w1, w2, bias.reshape(1, _FOUT))
    return out.reshape(_B, _V, _FOUT)
